# Initial kernel scaffold; baseline (speedup 1.0000x reference)
import jax
import jax.numpy as jnp
from jax.experimental import pallas as pl


def kernel(input_data, code, W_in, b_in, W_code, b_code, W_enc, b_enc, W_dec, b_dec, d_constraint):
    raise NotImplementedError("write your pallas kernel here")


# speedup vs baseline: 6.9168x; 6.9168x over previous
import jax
import jax.numpy as jnp
from jax import lax
from jax.experimental import pallas as pl
from jax.experimental.pallas import tpu as pltpu

L = 4096
D = 2048
H = 4096
CODE = 1024
K = 100
KP = 128

NEG = -3.4e38


def _topk_kernel(x_ref, vals_ref, idx_ref, xs):
    xs[...] = x_ref[...]
    vals_ref[...] = jnp.zeros_like(vals_ref)
    idx_ref[...] = jnp.zeros_like(idx_ref)
    br = x_ref.shape[0]
    lane = lax.broadcasted_iota(jnp.int32, (br, D), 1)
    kp_iota = lax.broadcasted_iota(jnp.int32, (br, KP), 1)

    def body(k, _):
        x = xs[...]
        m = jnp.max(x, axis=1, keepdims=True)
        first = jnp.min(jnp.where(x == m, lane, D), axis=1, keepdims=True)
        sel = kp_iota == k
        vals_ref[...] = jnp.where(sel, m, vals_ref[...])
        idx_ref[...] = jnp.where(sel, first, idx_ref[...])
        xs[...] = jnp.where(lane == first, NEG, x)
        return 0

    lax.fori_loop(0, K, body, 0)


def _mlp_kernel(vals_ref, w_in_ref, b_in_ref, code_ref, w_code_ref,
                b_code_ref, w_et_ref, w_eb_ref, b_enc_ref, e_ref, cvec):
    i = pl.program_id(0)

    @pl.when(i == 0)
    def _():
        ce = jnp.maximum(
            jnp.dot(code_ref[...], w_code_ref[...],
                    preferred_element_type=jnp.float32) + b_code_ref[...], 0.0)
        cvec[...] = jnp.dot(ce, w_eb_ref[...],
                            preferred_element_type=jnp.float32) + b_enc_ref[...]

    h = jnp.maximum(
        jnp.dot(vals_ref[...], w_in_ref[...],
                preferred_element_type=jnp.float32) + b_in_ref[...], 0.0)
    e_ref[...] = jnp.dot(h, w_et_ref[...],
                         preferred_element_type=jnp.float32) + cvec[...]


def _sample_kernel(x_ref, idx_ref, e_ref, g_ref, wdec_ref, bdec_ref,
                   out_ref, acc, xs):
    i = pl.program_id(0)
    n = pl.num_programs(0)
    br = x_ref.shape[0]
    lane = lax.broadcasted_iota(jnp.int32, (br, D), 1)

    @pl.when(i == 0)
    def _():
        acc[...] = jnp.zeros_like(acc)

    xs[...] = x_ref[...]
    kp_iota = lax.broadcasted_iota(jnp.int32, (br, KP), 1)

    def body(k, _):
        sel = kp_iota == k
        pos = jnp.sum(jnp.where(sel, idx_ref[...], 0), axis=1, keepdims=True)
        ev = jnp.sum(jnp.where(sel, e_ref[...], 0.0), axis=1, keepdims=True)
        xs[...] = jnp.where(lane == pos, ev, xs[...])
        return 0

    lax.fori_loop(0, K, body, 0)

    x = xs[...]
    m = jnp.max(x, axis=1, keepdims=True)
    ex = jnp.exp(x - m)
    z = jnp.sum(ex, axis=1, keepdims=True)
    y = jnp.log(ex / z + 1e-20) + g_ref[...]
    m2 = jnp.max(y, axis=1, keepdims=True)
    s = jnp.min(jnp.where(y == m2, lane, D), axis=1, keepdims=True)
    sf = s.astype(jnp.float32)
    acc[...] += jnp.sum(sf * wdec_ref[...], axis=0, keepdims=True)

    @pl.when(i == n - 1)
    def _():
        o = acc[...] + bdec_ref[...]
        nrm = jnp.sqrt(jnp.sum(o * o))
        out_ref[...] = o / jnp.maximum(nrm, 1e-12)


def kernel(input_data, code, W_in, b_in, W_code, b_code, W_enc, b_enc,
           W_dec, b_dec, d_constraint):
    del d_constraint
    f32 = jnp.float32

    w_in_pad = jnp.zeros((KP, H), f32).at[:K].set(W_in)
    w_et = jnp.zeros((H, KP), f32).at[:, :K].set(W_enc[:H])
    w_eb = jnp.zeros((H, KP), f32).at[:, :K].set(W_enc[H:])
    b_enc_pad = jnp.zeros((1, KP), f32).at[0, :K].set(b_enc)
    gumbel = jax.random.gumbel(jax.random.key(1234), (L, D), f32)

    BR1 = 256
    vals, idx = pl.pallas_call(
        _topk_kernel,
        grid=(L // BR1,),
        in_specs=[pl.BlockSpec((BR1, D), lambda i: (i, 0))],
        out_specs=[pl.BlockSpec((BR1, KP), lambda i: (i, 0)),
                   pl.BlockSpec((BR1, KP), lambda i: (i, 0))],
        out_shape=[jax.ShapeDtypeStruct((L, KP), f32),
                   jax.ShapeDtypeStruct((L, KP), jnp.int32)],
        scratch_shapes=[pltpu.VMEM((BR1, D), f32)],
    )(input_data)

    BR2 = 256
    e = pl.pallas_call(
        _mlp_kernel,
        grid=(L // BR2,),
        in_specs=[
            pl.BlockSpec((BR2, KP), lambda i: (i, 0)),
            pl.BlockSpec((KP, H), lambda i: (0, 0)),
            pl.BlockSpec((1, H), lambda i: (0, 0)),
            pl.BlockSpec((1, CODE), lambda i: (0, 0)),
            pl.BlockSpec((CODE, H), lambda i: (0, 0)),
            pl.BlockSpec((1, H), lambda i: (0, 0)),
            pl.BlockSpec((H, KP), lambda i: (0, 0)),
            pl.BlockSpec((H, KP), lambda i: (0, 0)),
            pl.BlockSpec((1, KP), lambda i: (0, 0)),
        ],
        out_specs=pl.BlockSpec((BR2, KP), lambda i: (i, 0)),
        out_shape=jax.ShapeDtypeStruct((L, KP), f32),
        scratch_shapes=[pltpu.VMEM((1, KP), f32)],
    )(vals, w_in_pad, b_in.reshape(1, H), code.reshape(1, CODE), W_code,
      b_code.reshape(1, H), w_et, w_eb, b_enc_pad)

    BR3 = 256
    out = pl.pallas_call(
        _sample_kernel,
        grid=(L // BR3,),
        in_specs=[
            pl.BlockSpec((BR3, D), lambda i: (i, 0)),
            pl.BlockSpec((BR3, KP), lambda i: (i, 0)),
            pl.BlockSpec((BR3, KP), lambda i: (i, 0)),
            pl.BlockSpec((BR3, D), lambda i: (i, 0)),
            pl.BlockSpec((BR3, CODE), lambda i: (i, 0)),
            pl.BlockSpec((1, CODE), lambda i: (0, 0)),
        ],
        out_specs=pl.BlockSpec((1, CODE), lambda i: (0, 0)),
        out_shape=jax.ShapeDtypeStruct((1, CODE), f32),
        scratch_shapes=[pltpu.VMEM((1, CODE), f32),
                        pltpu.VMEM((BR3, D), f32)],
    )(input_data, idx, e, gumbel, W_dec, b_dec.reshape(1, CODE))

    return out.reshape(CODE)


# speedup vs baseline: 9.1911x; 1.3288x over previous
import jax
import jax.numpy as jnp
from jax import lax
from jax.experimental import pallas as pl
from jax.experimental.pallas import tpu as pltpu
from jax.experimental.pallas import tpu_sc as plsc

L = 4096
D = 2048
H = 4096
CODE = 1024
K = 100
KP = 128

NEG = -3.4e38


def _topk_kernel(x_ref, vals_ref, idx_ref, xs):
    xs[...] = x_ref[...]
    vals_ref[...] = jnp.zeros_like(vals_ref)
    idx_ref[...] = jnp.zeros_like(idx_ref)
    br = x_ref.shape[0]
    lane = lax.broadcasted_iota(jnp.int32, (br, D), 1)
    kp_iota = lax.broadcasted_iota(jnp.int32, (br, KP), 1)

    def body(k, _):
        x = xs[...]
        m = jnp.max(x, axis=1, keepdims=True)
        first = jnp.min(jnp.where(x == m, lane, D), axis=1, keepdims=True)
        sel = kp_iota == k
        vals_ref[...] = jnp.where(sel, m, vals_ref[...])
        idx_ref[...] = jnp.where(sel, first, idx_ref[...])
        xs[...] = jnp.where(lane == first, NEG, x)
        return 0

    lax.fori_loop(0, K, body, 0)


def _mlp_kernel(vals_ref, w_in_ref, b_in_ref, code_ref, w_code_ref,
                b_code_ref, w_et_ref, w_eb_ref, b_enc_ref, e_ref, cvec):
    i = pl.program_id(0)

    @pl.when(i == 0)
    def _():
        ce = jnp.maximum(
            jnp.dot(code_ref[...], w_code_ref[...],
                    preferred_element_type=jnp.float32) + b_code_ref[...], 0.0)
        cvec[...] = jnp.dot(ce, w_eb_ref[...],
                            preferred_element_type=jnp.float32) + b_enc_ref[...]

    h = jnp.maximum(
        jnp.dot(vals_ref[...], w_in_ref[...],
                preferred_element_type=jnp.float32) + b_in_ref[...], 0.0)
    e_ref[...] = jnp.dot(h, w_et_ref[...],
                         preferred_element_type=jnp.float32) + cvec[...]


def _sc_scatter_kernel(x_hbm, idx_hbm, e_hbm, simu_hbm, idxbuf, ebuf, rows):
    wid = lax.axis_index("s") * 2 + lax.axis_index("c")
    rpw = L // 32
    base = wid * rpw
    pltpu.sync_copy(idx_hbm.at[pl.ds(base * KP, rpw * KP)], idxbuf)
    pltpu.sync_copy(e_hbm.at[pl.ds(base * KP, rpw * KP)], ebuf)
    ch = 32
    nch = rpw // ch
    iota16 = lax.broadcasted_iota(jnp.int32, (16,), 0)

    def chunk_body(c, _):
        row0 = base + c * ch
        pltpu.sync_copy(x_hbm.at[pl.ds(row0 * D, ch * D)], rows)

        def row_body(r, _):
            gr = c * ch + r
            roff = r * D
            for j in range(7):
                iv = idxbuf[pl.ds(gr * KP + j * 16, 16)] + roff
                ev = ebuf[pl.ds(gr * KP + j * 16, 16)]
                if j < 6:
                    plsc.store_scatter(rows, [iv], ev)
                else:
                    plsc.store_scatter(rows, [iv], ev,
                                       mask=iota16 < (K - 96))
            return 0

        lax.fori_loop(0, ch, row_body, 0)
        pltpu.sync_copy(rows, simu_hbm.at[pl.ds(row0 * D, ch * D)])
        return 0

    lax.fori_loop(0, nch, chunk_body, 0)


def _sample_kernel(x_ref, g_ref, wdec_ref, bdec_ref, out_ref, acc):
    i = pl.program_id(0)
    n = pl.num_programs(0)
    br = x_ref.shape[0]
    lane = lax.broadcasted_iota(jnp.int32, (br, D), 1)

    @pl.when(i == 0)
    def _():
        acc[...] = jnp.zeros_like(acc)

    x = x_ref[...]
    m = jnp.max(x, axis=1, keepdims=True)
    ex = jnp.exp(x - m)
    z = jnp.sum(ex, axis=1, keepdims=True)
    y = jnp.log(ex / z + 1e-20) + g_ref[...]
    m2 = jnp.max(y, axis=1, keepdims=True)
    s = jnp.min(jnp.where(y == m2, lane, D), axis=1, keepdims=True)
    sf = s.astype(jnp.float32)
    acc[...] += jnp.sum(sf * wdec_ref[...], axis=0, keepdims=True)

    @pl.when(i == n - 1)
    def _():
        o = acc[...] + bdec_ref[...]
        nrm = jnp.sqrt(jnp.sum(o * o))
        out_ref[...] = o / jnp.maximum(nrm, 1e-12)


def kernel(input_data, code, W_in, b_in, W_code, b_code, W_enc, b_enc,
           W_dec, b_dec, d_constraint):
    del d_constraint
    f32 = jnp.float32

    w_in_pad = jnp.zeros((KP, H), f32).at[:K].set(W_in)
    w_et = jnp.zeros((H, KP), f32).at[:, :K].set(W_enc[:H])
    w_eb = jnp.zeros((H, KP), f32).at[:, :K].set(W_enc[H:])
    b_enc_pad = jnp.zeros((1, KP), f32).at[0, :K].set(b_enc)
    gumbel = jax.random.gumbel(jax.random.key(1234), (L, D), f32)

    BR1 = 256
    vals, idx = pl.pallas_call(
        _topk_kernel,
        grid=(L // BR1,),
        in_specs=[pl.BlockSpec((BR1, D), lambda i: (i, 0))],
        out_specs=[pl.BlockSpec((BR1, KP), lambda i: (i, 0)),
                   pl.BlockSpec((BR1, KP), lambda i: (i, 0))],
        out_shape=[jax.ShapeDtypeStruct((L, KP), f32),
                   jax.ShapeDtypeStruct((L, KP), jnp.int32)],
        scratch_shapes=[pltpu.VMEM((BR1, D), f32)],
    )(input_data)

    BR2 = 256
    e = pl.pallas_call(
        _mlp_kernel,
        grid=(L // BR2,),
        in_specs=[
            pl.BlockSpec((BR2, KP), lambda i: (i, 0)),
            pl.BlockSpec((KP, H), lambda i: (0, 0)),
            pl.BlockSpec((1, H), lambda i: (0, 0)),
            pl.BlockSpec((1, CODE), lambda i: (0, 0)),
            pl.BlockSpec((CODE, H), lambda i: (0, 0)),
            pl.BlockSpec((1, H), lambda i: (0, 0)),
            pl.BlockSpec((H, KP), lambda i: (0, 0)),
            pl.BlockSpec((H, KP), lambda i: (0, 0)),
            pl.BlockSpec((1, KP), lambda i: (0, 0)),
        ],
        out_specs=pl.BlockSpec((BR2, KP), lambda i: (i, 0)),
        out_shape=jax.ShapeDtypeStruct((L, KP), f32),
        scratch_shapes=[pltpu.VMEM((1, KP), f32)],
    )(vals, w_in_pad, b_in.reshape(1, H), code.reshape(1, CODE), W_code,
      b_code.reshape(1, H), w_et, w_eb, b_enc_pad)

    simu = pl.kernel(
        _sc_scatter_kernel,
        out_type=jax.ShapeDtypeStruct((L * D,), f32),
        mesh=plsc.VectorSubcoreMesh(core_axis_name="c", subcore_axis_name="s"),
        scratch_types=[
            pltpu.VMEM(((L // 32) * KP,), jnp.int32),
            pltpu.VMEM(((L // 32) * KP,), f32),
            pltpu.VMEM((32 * D,), f32),
        ],
        compiler_params=pltpu.CompilerParams(
            needs_layout_passes=False, use_tc_tiling_on_sc=False),
    )(input_data.reshape(L * D), idx.reshape(L * KP), e.reshape(L * KP))
    simu = simu.reshape(L, D)

    BR3 = 256
    out = pl.pallas_call(
        _sample_kernel,
        grid=(L // BR3,),
        in_specs=[
            pl.BlockSpec((BR3, D), lambda i: (i, 0)),
            pl.BlockSpec((BR3, D), lambda i: (i, 0)),
            pl.BlockSpec((BR3, CODE), lambda i: (i, 0)),
            pl.BlockSpec((1, CODE), lambda i: (0, 0)),
        ],
        out_specs=pl.BlockSpec((1, CODE), lambda i: (0, 0)),
        out_shape=jax.ShapeDtypeStruct((1, CODE), f32),
        scratch_shapes=[pltpu.VMEM((1, CODE), f32)],
    )(simu, gumbel, W_dec, b_dec.reshape(1, CODE))

    return out.reshape(CODE)


# speedup vs baseline: 9.5282x; 1.0367x over previous
import jax
import jax.numpy as jnp
from jax import lax
from jax.experimental import pallas as pl
from jax.experimental.pallas import tpu as pltpu
from jax.experimental.pallas import tpu_sc as plsc

L = 4096
D = 2048
H = 4096
CODE = 1024
K = 100
KP = 128

NW = 32
RPW = L // NW

NEG = -3.4e38
XOR = 0x7FFFFFFF
INT_MIN = -2147483648


def _bisect_kernel(x_ref, tkey_ref, m_ref):
    x = x_ref[...]
    bu = lax.bitcast_convert_type(x, jnp.int32)
    ks = jnp.where(bu >= 0, bu, bu ^ jnp.int32(XOR))
    cnt0 = jnp.sum((ks >= 0).astype(jnp.int32), axis=1, keepdims=True)
    p = jnp.where(cnt0 >= K, jnp.int32(0), jnp.int32(INT_MIN))
    for b in range(30, -1, -1):
        cand = p + jnp.int32(1 << b)
        cnt = jnp.sum((ks >= cand).astype(jnp.int32), axis=1, keepdims=True)
        p = jnp.where(cnt >= K, cand, p)
    a = jnp.sum((ks > p).astype(jnp.int32), axis=1, keepdims=True)
    tkey_ref[...] = p
    m_ref[...] = K - a


def _sc_compact_kernel(x_hbm, tkey_hbm, m_hbm, vals_hbm, idx_hbm,
                       rowbuf, tkbuf, mbuf, ovals, oidx, pcg, pce):
    wid = lax.axis_index("s") * 2 + lax.axis_index("c")
    base = wid * RPW
    pltpu.sync_copy(tkey_hbm.at[pl.ds(base, RPW)], tkbuf)
    pltpu.sync_copy(m_hbm.at[pl.ds(base, RPW)], mbuf)
    ch = 16
    nch = RPW // ch
    iota16 = lax.broadcasted_iota(jnp.int32, (16,), 0)

    def chunk_body(c, _):
        pltpu.sync_copy(x_hbm.at[pl.ds((base + c * ch) * D, ch * D)], rowbuf)

        def row_body(r, _):
            gr = c * ch + r
            roff = r * D
            obase = gr * KP
            grv = jnp.full((16,), gr, jnp.int32)
            tv = plsc.load_gather(tkbuf, [grv])
            av = K - plsc.load_gather(mbuf, [grv])
            lane0 = iota16 == 0

            for jj in range(8):
                ovals[pl.ds(obase + jj * 16, 16)] = jnp.full((16,), NEG,
                                                             jnp.float32)
                oidx[pl.ds(obase + jj * 16, 16)] = iota16 + (4000 + jj * 16)

            def pass_a(j, _):
                v = rowbuf[pl.ds(roff + j * 16, 16)]
                bu = plsc.bitcast(v, jnp.int32)
                ks = jnp.where(bu >= 0, bu, bu ^ jnp.int32(XOR))
                gt = (ks > tv).astype(jnp.int32)
                eq = (ks == tv).astype(jnp.int32)
                jv = jnp.full((16,), j, jnp.int32)
                plsc.store_scatter(pcg, [jv],
                                   jnp.full((16,), jnp.sum(gt), jnp.int32),
                                   mask=lane0)
                plsc.store_scatter(pce, [jv],
                                   jnp.full((16,), jnp.sum(eq), jnp.int32),
                                   mask=lane0)
                return 0

            lax.fori_loop(0, KP, pass_a, 0)

            def prefix(j8, carries):
                cg, ce = carries
                vg = pcg[pl.ds(j8 * 16, 16)]
                ve = pce[pl.ds(j8 * 16, 16)]
                pcg[pl.ds(j8 * 16, 16)] = plsc.cumsum(vg) - vg + cg
                pce[pl.ds(j8 * 16, 16)] = plsc.cumsum(ve) - ve + ce
                return (cg + jnp.sum(vg), ce + jnp.sum(ve))

            lax.fori_loop(0, 8, prefix, (jnp.int32(0), jnp.int32(0)))

            def pass_b(j, _):
                v = rowbuf[pl.ds(roff + j * 16, 16)]
                bu = plsc.bitcast(v, jnp.int32)
                ks = jnp.where(bu >= 0, bu, bu ^ jnp.int32(XOR))
                gt = ks > tv
                eq = ks == tv
                iv = iota16 + j * 16
                jv = jnp.full((16,), j, jnp.int32)
                gti = gt.astype(jnp.int32)
                exg = plsc.cumsum(gti) - gti
                tgt = exg + plsc.load_gather(pcg, [jv]) + obase
                plsc.store_scatter(ovals, [tgt], v, mask=gt)
                plsc.store_scatter(oidx, [tgt], iv, mask=gt)
                eqi = eq.astype(jnp.int32)
                exe = plsc.cumsum(eqi) - eqi
                pose = exe + av + plsc.load_gather(pce, [jv])
                keep = eq & (pose < K)
                tgte = pose + obase
                plsc.store_scatter(ovals, [tgte], v, mask=keep)
                plsc.store_scatter(oidx, [tgte], iv, mask=keep)
                return 0

            lax.fori_loop(0, KP, pass_b, 0)
            return 0

        lax.fori_loop(0, ch, row_body, 0)
        return 0

    lax.fori_loop(0, nch, chunk_body, 0)
    pltpu.sync_copy(ovals, vals_hbm.at[pl.ds(base * KP, RPW * KP)])
    pltpu.sync_copy(oidx, idx_hbm.at[pl.ds(base * KP, RPW * KP)])


def _rank_mlp_kernel(cv_ref, ci_ref, w_in_ref, b_in_ref, code_ref, w_code_ref,
                     b_code_ref, w_et_ref, w_eb_ref, b_enc_ref,
                     e_ref, sidx_ref, cvec):
    i = pl.program_id(0)

    @pl.when(i == 0)
    def _():
        ce = jnp.maximum(
            jnp.dot(code_ref[...], w_code_ref[...],
                    preferred_element_type=jnp.float32) + b_code_ref[...], 0.0)
        cvec[...] = jnp.dot(ce, w_eb_ref[...],
                            preferred_element_type=jnp.float32) + b_enc_ref[...]

    v = cv_ref[...]
    ix = ci_ref[...]
    va = v[:, :, None]
    vb = v[:, None, :]
    ia = ix[:, :, None]
    ib = ix[:, None, :]
    cmp = (va > vb) | ((va == vb) & (ia < ib))
    rank = jnp.sum(cmp.astype(jnp.int32), axis=1)
    br = v.shape[0]
    kio3 = lax.broadcasted_iota(jnp.int32, (br, KP, KP), 2)
    oneh = rank[:, :, None] == kio3
    sv = jnp.sum(jnp.where(oneh, va, 0.0), axis=1)
    si = jnp.sum(jnp.where(oneh, ia, 0), axis=1)
    kio2 = lax.broadcasted_iota(jnp.int32, (br, KP), 1)
    sv = jnp.where(kio2 < K, sv, 0.0)
    sidx_ref[...] = si
    h = jnp.maximum(
        jnp.dot(sv, w_in_ref[...],
                preferred_element_type=jnp.float32) + b_in_ref[...], 0.0)
    e_ref[...] = jnp.dot(h, w_et_ref[...],
                         preferred_element_type=jnp.float32) + cvec[...]


def _sc_scatter_kernel(x_hbm, idx_hbm, e_hbm, simu_hbm, idxbuf, ebuf, rows):
    wid = lax.axis_index("s") * 2 + lax.axis_index("c")
    base = wid * RPW
    pltpu.sync_copy(idx_hbm.at[pl.ds(base * KP, RPW * KP)], idxbuf)
    pltpu.sync_copy(e_hbm.at[pl.ds(base * KP, RPW * KP)], ebuf)
    ch = 32
    nch = RPW // ch
    iota16 = lax.broadcasted_iota(jnp.int32, (16,), 0)

    def chunk_body(c, _):
        row0 = base + c * ch
        pltpu.sync_copy(x_hbm.at[pl.ds(row0 * D, ch * D)], rows)

        def row_body(r, _):
            gr = c * ch + r
            roff = r * D
            for j in range(7):
                iv = idxbuf[pl.ds(gr * KP + j * 16, 16)] + roff
                ev = ebuf[pl.ds(gr * KP + j * 16, 16)]
                if j < 6:
                    plsc.store_scatter(rows, [iv], ev)
                else:
                    plsc.store_scatter(rows, [iv], ev,
                                       mask=iota16 < (K - 96))
            return 0

        lax.fori_loop(0, ch, row_body, 0)
        pltpu.sync_copy(rows, simu_hbm.at[pl.ds(row0 * D, ch * D)])
        return 0

    lax.fori_loop(0, nch, chunk_body, 0)


def _sample_kernel(x_ref, g_ref, wdec_ref, bdec_ref, out_ref, acc):
    i = pl.program_id(0)
    n = pl.num_programs(0)
    br = x_ref.shape[0]
    lane = lax.broadcasted_iota(jnp.int32, (br, D), 1)

    @pl.when(i == 0)
    def _():
        acc[...] = jnp.zeros_like(acc)

    x = x_ref[...]
    m = jnp.max(x, axis=1, keepdims=True)
    ex = jnp.exp(x - m)
    z = jnp.sum(ex, axis=1, keepdims=True)
    y = jnp.log(ex / z + 1e-20) + g_ref[...]
    m2 = jnp.max(y, axis=1, keepdims=True)
    s = jnp.min(jnp.where(y == m2, lane, D), axis=1, keepdims=True)
    sf = s.astype(jnp.float32)
    acc[...] += jnp.sum(sf * wdec_ref[...], axis=0, keepdims=True)

    @pl.when(i == n - 1)
    def _():
        o = acc[...] + bdec_ref[...]
        nrm = jnp.sqrt(jnp.sum(o * o))
        out_ref[...] = o / jnp.maximum(nrm, 1e-12)


_SC_PARAMS = pltpu.CompilerParams(needs_layout_passes=False,
                                  use_tc_tiling_on_sc=False)
_SC_MESH = dict(core_axis_name="c", subcore_axis_name="s")


def kernel(input_data, code, W_in, b_in, W_code, b_code, W_enc, b_enc,
           W_dec, b_dec, d_constraint):
    del d_constraint
    f32 = jnp.float32
    i32 = jnp.int32

    w_in_pad = jnp.zeros((KP, H), f32).at[:K].set(W_in)
    w_et = jnp.zeros((H, KP), f32).at[:, :K].set(W_enc[:H])
    w_eb = jnp.zeros((H, KP), f32).at[:, :K].set(W_enc[H:])
    b_enc_pad = jnp.zeros((1, KP), f32).at[0, :K].set(b_enc)
    gumbel = jax.random.gumbel(jax.random.key(1234), (L, D), f32)
    x_flat = input_data.reshape(L * D)

    BRB = 256
    tkey, mm = pl.pallas_call(
        _bisect_kernel,
        grid=(L // BRB,),
        in_specs=[pl.BlockSpec((BRB, D), lambda i: (i, 0))],
        out_specs=[pl.BlockSpec((BRB, 1), lambda i: (i, 0)),
                   pl.BlockSpec((BRB, 1), lambda i: (i, 0))],
        out_shape=[jax.ShapeDtypeStruct((L, 1), i32),
                   jax.ShapeDtypeStruct((L, 1), i32)],
    )(input_data)

    cv_flat, ci_flat = pl.kernel(
        _sc_compact_kernel,
        out_type=[jax.ShapeDtypeStruct((L * KP,), f32),
                  jax.ShapeDtypeStruct((L * KP,), i32)],
        mesh=plsc.VectorSubcoreMesh(**_SC_MESH),
        scratch_types=[
            pltpu.VMEM((16 * D,), f32),
            pltpu.VMEM((RPW,), i32),
            pltpu.VMEM((RPW,), i32),
            pltpu.VMEM((RPW * KP,), f32),
            pltpu.VMEM((RPW * KP,), i32),
            pltpu.VMEM((KP,), i32),
            pltpu.VMEM((KP,), i32),
        ],
        compiler_params=_SC_PARAMS,
    )(x_flat, tkey.reshape(L), mm.reshape(L))
    cv = cv_flat.reshape(L, KP)
    ci = ci_flat.reshape(L, KP)

    BR2 = 64
    e, sidx = pl.pallas_call(
        _rank_mlp_kernel,
        grid=(L // BR2,),
        in_specs=[
            pl.BlockSpec((BR2, KP), lambda i: (i, 0)),
            pl.BlockSpec((BR2, KP), lambda i: (i, 0)),
            pl.BlockSpec((KP, H), lambda i: (0, 0)),
            pl.BlockSpec((1, H), lambda i: (0, 0)),
            pl.BlockSpec((1, CODE), lambda i: (0, 0)),
            pl.BlockSpec((CODE, H), lambda i: (0, 0)),
            pl.BlockSpec((1, H), lambda i: (0, 0)),
            pl.BlockSpec((H, KP), lambda i: (0, 0)),
            pl.BlockSpec((H, KP), lambda i: (0, 0)),
            pl.BlockSpec((1, KP), lambda i: (0, 0)),
        ],
        out_specs=[pl.BlockSpec((BR2, KP), lambda i: (i, 0)),
                   pl.BlockSpec((BR2, KP), lambda i: (i, 0))],
        out_shape=[jax.ShapeDtypeStruct((L, KP), f32),
                   jax.ShapeDtypeStruct((L, KP), i32)],
        scratch_shapes=[pltpu.VMEM((1, KP), f32)],
    )(cv, ci, w_in_pad, b_in.reshape(1, H), code.reshape(1, CODE), W_code,
      b_code.reshape(1, H), w_et, w_eb, b_enc_pad)

    simu = pl.kernel(
        _sc_scatter_kernel,
        out_type=jax.ShapeDtypeStruct((L * D,), f32),
        mesh=plsc.VectorSubcoreMesh(**_SC_MESH),
        scratch_types=[
            pltpu.VMEM((RPW * KP,), i32),
            pltpu.VMEM((RPW * KP,), f32),
            pltpu.VMEM((32 * D,), f32),
        ],
        compiler_params=_SC_PARAMS,
    )(x_flat, sidx.reshape(L * KP), e.reshape(L * KP))
    simu = simu.reshape(L, D)

    BR3 = 256
    out = pl.pallas_call(
        _sample_kernel,
        grid=(L // BR3,),
        in_specs=[
            pl.BlockSpec((BR3, D), lambda i: (i, 0)),
            pl.BlockSpec((BR3, D), lambda i: (i, 0)),
            pl.BlockSpec((BR3, CODE), lambda i: (i, 0)),
            pl.BlockSpec((1, CODE), lambda i: (0, 0)),
        ],
        out_specs=pl.BlockSpec((1, CODE), lambda i: (0, 0)),
        out_shape=jax.ShapeDtypeStruct((1, CODE), f32),
        scratch_shapes=[pltpu.VMEM((1, CODE), f32)],
    )(simu, gumbel, W_dec, b_dec.reshape(1, CODE))

    return out.reshape(CODE)


# speedup vs baseline: 13.1325x; 1.3783x over previous
import jax
import jax.numpy as jnp
from jax import lax
from jax.experimental import pallas as pl
from jax.experimental.pallas import tpu as pltpu
from jax.experimental.pallas import tpu_sc as plsc

L = 4096
D = 2048
H = 4096
CODE = 1024
K = 100
KP = 128

NW = 32
RPW = L // NW

NEG = -3.4e38
XOR = 0x7FFFFFFF
INT_MIN = -2147483648


def _bisect_kernel(x_ref, tkey_ref, m_ref):
    x = x_ref[...]
    bu = lax.bitcast_convert_type(x, jnp.int32)
    ks = jnp.where(bu >= 0, bu, bu ^ jnp.int32(XOR))
    cnt0 = jnp.sum((ks >= 0).astype(jnp.int32), axis=1, keepdims=True)
    p = jnp.where(cnt0 >= K, jnp.int32(0), jnp.int32(INT_MIN))
    for b in range(30, -1, -1):
        cand = p + jnp.int32(1 << b)
        cnt = jnp.sum((ks >= cand).astype(jnp.int32), axis=1, keepdims=True)
        p = jnp.where(cnt >= K, cand, p)
    a = jnp.sum((ks > p).astype(jnp.int32), axis=1, keepdims=True)
    tkey_ref[...] = p
    m_ref[...] = K - a


def _sc_compact_kernel(x_hbm, tkey_hbm, m_hbm, vals_hbm, idx_hbm,
                       rowbuf, tkbuf, mbuf, ovals, oidx):
    wid = lax.axis_index("s") * 2 + lax.axis_index("c")
    base = wid * RPW
    pltpu.sync_copy(tkey_hbm.at[pl.ds(base, RPW)], tkbuf)
    pltpu.sync_copy(m_hbm.at[pl.ds(base, RPW)], mbuf)
    ch = 16
    nch = RPW // ch
    iota16 = lax.broadcasted_iota(jnp.int32, (16,), 0)

    def chunk_body(c, _):
        pltpu.sync_copy(x_hbm.at[pl.ds((base + c * ch) * D, ch * D)], rowbuf)

        def row_body(r, _):
            gr = c * ch + r
            roff = r * D
            obase = gr * KP
            grv = jnp.full((16,), gr, jnp.int32)
            tv = plsc.load_gather(tkbuf, [grv])
            av = K - plsc.load_gather(mbuf, [grv])

            for jj in range(8):
                ovals[pl.ds(obase + jj * 16, 16)] = jnp.full((16,), NEG,
                                                             jnp.float32)
                oidx[pl.ds(obase + jj * 16, 16)] = iota16 + (4000 + jj * 16)

            cg0 = jnp.full((16,), obase, jnp.int32)
            ce0 = cg0 + av
            limit = obase + K

            def scan_vregs(j, carries):
                cg, ce = carries
                v = rowbuf[pl.ds(roff + j * 16, 16)]
                bu = plsc.bitcast(v, jnp.int32)
                ks = jnp.where(bu >= 0, bu, bu ^ jnp.int32(XOR))
                gt = ks > tv
                eq = ks == tv
                iv = iota16 + j * 16
                gti = gt.astype(jnp.int32)
                tgt = cg + plsc.cumsum(gti) - gti
                plsc.store_scatter(ovals, [tgt], v, mask=gt)
                plsc.store_scatter(oidx, [tgt], iv, mask=gt)
                eqi = eq.astype(jnp.int32)
                pose = ce + plsc.cumsum(eqi) - eqi
                keep = eq & (pose < limit)
                plsc.store_scatter(ovals, [pose], v, mask=keep)
                plsc.store_scatter(oidx, [pose], iv, mask=keep)
                return (cg + plsc.all_reduce_population_count(gt),
                        ce + plsc.all_reduce_population_count(eq))

            lax.fori_loop(0, KP, scan_vregs, (cg0, ce0))
            return 0

        lax.fori_loop(0, ch, row_body, 0)
        return 0

    lax.fori_loop(0, nch, chunk_body, 0)
    pltpu.sync_copy(ovals, vals_hbm.at[pl.ds(base * KP, RPW * KP)])
    pltpu.sync_copy(oidx, idx_hbm.at[pl.ds(base * KP, RPW * KP)])


def _rank_mlp_kernel(cv_ref, ci_ref, w_in_ref, b_in_ref, code_ref, w_code_ref,
                     b_code_ref, w_et_ref, w_eb_ref, b_enc_ref,
                     e_ref, sidx_ref, cvec):
    i = pl.program_id(0)

    @pl.when(i == 0)
    def _():
        ce = jnp.maximum(
            jnp.dot(code_ref[...], w_code_ref[...],
                    preferred_element_type=jnp.float32) + b_code_ref[...], 0.0)
        cvec[...] = jnp.dot(ce, w_eb_ref[...],
                            preferred_element_type=jnp.float32) + b_enc_ref[...]

    v = cv_ref[...]
    ix = ci_ref[...]
    va = v[:, :, None]
    vb = v[:, None, :]
    ia = ix[:, :, None]
    ib = ix[:, None, :]
    cmp = (va > vb) | ((va == vb) & (ia < ib))
    rank = jnp.sum(cmp.astype(jnp.int32), axis=1)
    br = v.shape[0]
    kio3 = lax.broadcasted_iota(jnp.int32, (br, KP, KP), 2)
    oneh = rank[:, :, None] == kio3
    sv = jnp.sum(jnp.where(oneh, va, 0.0), axis=1)
    si = jnp.sum(jnp.where(oneh, ia, 0), axis=1)
    kio2 = lax.broadcasted_iota(jnp.int32, (br, KP), 1)
    sv = jnp.where(kio2 < K, sv, 0.0)
    sidx_ref[...] = si
    h = jnp.maximum(
        jnp.dot(sv, w_in_ref[...],
                preferred_element_type=jnp.float32) + b_in_ref[...], 0.0)
    e_ref[...] = jnp.dot(h, w_et_ref[...],
                         preferred_element_type=jnp.float32) + cvec[...]


def _sc_scatter_kernel(x_hbm, idx_hbm, e_hbm, simu_hbm, idxbuf, ebuf, rows):
    wid = lax.axis_index("s") * 2 + lax.axis_index("c")
    base = wid * RPW
    pltpu.sync_copy(idx_hbm.at[pl.ds(base * KP, RPW * KP)], idxbuf)
    pltpu.sync_copy(e_hbm.at[pl.ds(base * KP, RPW * KP)], ebuf)
    ch = 32
    nch = RPW // ch
    iota16 = lax.broadcasted_iota(jnp.int32, (16,), 0)

    def chunk_body(c, _):
        row0 = base + c * ch
        pltpu.sync_copy(x_hbm.at[pl.ds(row0 * D, ch * D)], rows)

        def row_body(r, _):
            gr = c * ch + r
            roff = r * D
            for j in range(7):
                iv = idxbuf[pl.ds(gr * KP + j * 16, 16)] + roff
                ev = ebuf[pl.ds(gr * KP + j * 16, 16)]
                if j < 6:
                    plsc.store_scatter(rows, [iv], ev)
                else:
                    plsc.store_scatter(rows, [iv], ev,
                                       mask=iota16 < (K - 96))
            return 0

        lax.fori_loop(0, ch, row_body, 0)
        pltpu.sync_copy(rows, simu_hbm.at[pl.ds(row0 * D, ch * D)])
        return 0

    lax.fori_loop(0, nch, chunk_body, 0)


def _sample_kernel(x_ref, g_ref, wdec_ref, bdec_ref, out_ref, acc):
    i = pl.program_id(0)
    n = pl.num_programs(0)
    br = x_ref.shape[0]
    lane = lax.broadcasted_iota(jnp.int32, (br, D), 1)

    @pl.when(i == 0)
    def _():
        acc[...] = jnp.zeros_like(acc)

    x = x_ref[...]
    m = jnp.max(x, axis=1, keepdims=True)
    ex = jnp.exp(x - m)
    z = jnp.sum(ex, axis=1, keepdims=True)
    y = jnp.log(ex / z + 1e-20) + g_ref[...]
    m2 = jnp.max(y, axis=1, keepdims=True)
    s = jnp.min(jnp.where(y == m2, lane, D), axis=1, keepdims=True)
    sf = s.astype(jnp.float32)
    acc[...] += jnp.sum(sf * wdec_ref[...], axis=0, keepdims=True)

    @pl.when(i == n - 1)
    def _():
        o = acc[...] + bdec_ref[...]
        nrm = jnp.sqrt(jnp.sum(o * o))
        out_ref[...] = o / jnp.maximum(nrm, 1e-12)


_SC_PARAMS = pltpu.CompilerParams(needs_layout_passes=False,
                                  use_tc_tiling_on_sc=False)
_SC_MESH = dict(core_axis_name="c", subcore_axis_name="s")


def kernel(input_data, code, W_in, b_in, W_code, b_code, W_enc, b_enc,
           W_dec, b_dec, d_constraint):
    del d_constraint
    f32 = jnp.float32
    i32 = jnp.int32

    w_in_pad = jnp.zeros((KP, H), f32).at[:K].set(W_in)
    w_et = jnp.zeros((H, KP), f32).at[:, :K].set(W_enc[:H])
    w_eb = jnp.zeros((H, KP), f32).at[:, :K].set(W_enc[H:])
    b_enc_pad = jnp.zeros((1, KP), f32).at[0, :K].set(b_enc)
    gumbel = jax.random.gumbel(jax.random.key(1234), (L, D), f32)
    x_flat = input_data.reshape(L * D)

    BRB = 256
    tkey, mm = pl.pallas_call(
        _bisect_kernel,
        grid=(L // BRB,),
        in_specs=[pl.BlockSpec((BRB, D), lambda i: (i, 0))],
        out_specs=[pl.BlockSpec((BRB, 1), lambda i: (i, 0)),
                   pl.BlockSpec((BRB, 1), lambda i: (i, 0))],
        out_shape=[jax.ShapeDtypeStruct((L, 1), i32),
                   jax.ShapeDtypeStruct((L, 1), i32)],
    )(input_data)

    cv_flat, ci_flat = pl.kernel(
        _sc_compact_kernel,
        out_type=[jax.ShapeDtypeStruct((L * KP,), f32),
                  jax.ShapeDtypeStruct((L * KP,), i32)],
        mesh=plsc.VectorSubcoreMesh(**_SC_MESH),
        scratch_types=[
            pltpu.VMEM((16 * D,), f32),
            pltpu.VMEM((RPW,), i32),
            pltpu.VMEM((RPW,), i32),
            pltpu.VMEM((RPW * KP,), f32),
            pltpu.VMEM((RPW * KP,), i32),
        ],
        compiler_params=_SC_PARAMS,
    )(x_flat, tkey.reshape(L), mm.reshape(L))
    cv = cv_flat.reshape(L, KP)
    ci = ci_flat.reshape(L, KP)

    BR2 = 64
    e, sidx = pl.pallas_call(
        _rank_mlp_kernel,
        grid=(L // BR2,),
        in_specs=[
            pl.BlockSpec((BR2, KP), lambda i: (i, 0)),
            pl.BlockSpec((BR2, KP), lambda i: (i, 0)),
            pl.BlockSpec((KP, H), lambda i: (0, 0)),
            pl.BlockSpec((1, H), lambda i: (0, 0)),
            pl.BlockSpec((1, CODE), lambda i: (0, 0)),
            pl.BlockSpec((CODE, H), lambda i: (0, 0)),
            pl.BlockSpec((1, H), lambda i: (0, 0)),
            pl.BlockSpec((H, KP), lambda i: (0, 0)),
            pl.BlockSpec((H, KP), lambda i: (0, 0)),
            pl.BlockSpec((1, KP), lambda i: (0, 0)),
        ],
        out_specs=[pl.BlockSpec((BR2, KP), lambda i: (i, 0)),
                   pl.BlockSpec((BR2, KP), lambda i: (i, 0))],
        out_shape=[jax.ShapeDtypeStruct((L, KP), f32),
                   jax.ShapeDtypeStruct((L, KP), i32)],
        scratch_shapes=[pltpu.VMEM((1, KP), f32)],
    )(cv, ci, w_in_pad, b_in.reshape(1, H), code.reshape(1, CODE), W_code,
      b_code.reshape(1, H), w_et, w_eb, b_enc_pad)

    simu = pl.kernel(
        _sc_scatter_kernel,
        out_type=jax.ShapeDtypeStruct((L * D,), f32),
        mesh=plsc.VectorSubcoreMesh(**_SC_MESH),
        scratch_types=[
            pltpu.VMEM((RPW * KP,), i32),
            pltpu.VMEM((RPW * KP,), f32),
            pltpu.VMEM((32 * D,), f32),
        ],
        compiler_params=_SC_PARAMS,
    )(x_flat, sidx.reshape(L * KP), e.reshape(L * KP))
    simu = simu.reshape(L, D)

    BR3 = 256
    out = pl.pallas_call(
        _sample_kernel,
        grid=(L // BR3,),
        in_specs=[
            pl.BlockSpec((BR3, D), lambda i: (i, 0)),
            pl.BlockSpec((BR3, D), lambda i: (i, 0)),
            pl.BlockSpec((BR3, CODE), lambda i: (i, 0)),
            pl.BlockSpec((1, CODE), lambda i: (0, 0)),
        ],
        out_specs=pl.BlockSpec((1, CODE), lambda i: (0, 0)),
        out_shape=jax.ShapeDtypeStruct((1, CODE), f32),
        scratch_shapes=[pltpu.VMEM((1, CODE), f32)],
    )(simu, gumbel, W_dec, b_dec.reshape(1, CODE))

    return out.reshape(CODE)


# speedup vs baseline: 14.2974x; 1.0887x over previous
import jax
import jax.numpy as jnp
from jax import lax
from jax.experimental import pallas as pl
from jax.experimental.pallas import tpu as pltpu
from jax.experimental.pallas import tpu_sc as plsc

L = 4096
D = 2048
H = 4096
CODE = 1024
K = 100
KP = 128

NW = 32
RPW = L // NW

NEG = -3.4e38
XOR = 0x7FFFFFFF
INT_MIN = -2147483648


def _bisect_kernel(x_ref, tkey_ref, m_ref):
    x = x_ref[...]
    bu = lax.bitcast_convert_type(x, jnp.int32)
    ks = jnp.where(bu >= 0, bu, bu ^ jnp.int32(XOR))
    cnt0 = jnp.sum((ks >= 0).astype(jnp.int32), axis=1, keepdims=True)
    p = jnp.where(cnt0 >= K, jnp.int32(0), jnp.int32(INT_MIN))
    for b in range(30, -1, -1):
        cand = p + jnp.int32(1 << b)
        cnt = jnp.sum((ks >= cand).astype(jnp.int32), axis=1, keepdims=True)
        p = jnp.where(cnt >= K, cand, p)
    a = jnp.sum((ks > p).astype(jnp.int32), axis=1, keepdims=True)
    tkey_ref[...] = p
    m_ref[...] = K - a


def _sc_compact_kernel(x_hbm, tkey_hbm, m_hbm, vals_hbm, idx_hbm,
                       rowbuf, tkbuf, mbuf, ovals, oidx):
    wid = lax.axis_index("s") * 2 + lax.axis_index("c")
    base = wid * RPW
    pltpu.sync_copy(tkey_hbm.at[pl.ds(base, RPW)], tkbuf)
    pltpu.sync_copy(m_hbm.at[pl.ds(base, RPW)], mbuf)
    ch = 16
    nch = RPW // ch
    iota16 = lax.broadcasted_iota(jnp.int32, (16,), 0)

    def chunk_body(c, _):
        pltpu.sync_copy(x_hbm.at[pl.ds((base + c * ch) * D, ch * D)], rowbuf)

        def row_body(r, _):
            gr = c * ch + r
            roff = r * D
            obase = gr * KP
            grv = jnp.full((16,), gr, jnp.int32)
            tv = plsc.load_gather(tkbuf, [grv])
            av = K - plsc.load_gather(mbuf, [grv])

            for jj in range(8):
                ovals[pl.ds(obase + jj * 16, 16)] = jnp.full((16,), NEG,
                                                             jnp.float32)
                oidx[pl.ds(obase + jj * 16, 16)] = iota16 + (4000 + jj * 16)

            cg0 = jnp.full((16,), obase, jnp.int32)
            ce0 = cg0 + av
            limit = obase + K

            def scan_vregs(j, carries):
                cg, ce = carries
                v = rowbuf[pl.ds(roff + j * 16, 16)]
                bu = plsc.bitcast(v, jnp.int32)
                ks = jnp.where(bu >= 0, bu, bu ^ jnp.int32(XOR))
                gt = ks > tv
                eq = ks == tv
                iv = iota16 + j * 16
                gti = gt.astype(jnp.int32)
                tgt = cg + plsc.cumsum(gti) - gti
                plsc.store_scatter(ovals, [tgt], v, mask=gt)
                plsc.store_scatter(oidx, [tgt], iv, mask=gt)
                eqi = eq.astype(jnp.int32)
                pose = ce + plsc.cumsum(eqi) - eqi
                keep = eq & (pose < limit)
                plsc.store_scatter(ovals, [pose], v, mask=keep)
                plsc.store_scatter(oidx, [pose], iv, mask=keep)
                return (cg + plsc.all_reduce_population_count(gt),
                        ce + plsc.all_reduce_population_count(eq))

            lax.fori_loop(0, KP, scan_vregs, (cg0, ce0), unroll=8)
            return 0

        lax.fori_loop(0, ch, row_body, 0)
        return 0

    lax.fori_loop(0, nch, chunk_body, 0)
    pltpu.sync_copy(ovals, vals_hbm.at[pl.ds(base * KP, RPW * KP)])
    pltpu.sync_copy(oidx, idx_hbm.at[pl.ds(base * KP, RPW * KP)])


def _rank_mlp_kernel(cv_ref, w_in_ref, b_in_ref, code_ref, w_code_ref,
                     b_code_ref, w_et_ref, w_eb_ref, b_enc_ref,
                     e_ref, rank_ref, cvec):
    i = pl.program_id(0)

    @pl.when(i == 0)
    def _():
        ce = jnp.maximum(
            jnp.dot(code_ref[...], w_code_ref[...],
                    preferred_element_type=jnp.float32) + b_code_ref[...], 0.0)
        cvec[...] = jnp.dot(ce, w_eb_ref[...],
                            preferred_element_type=jnp.float32) + b_enc_ref[...]

    v = cv_ref[...]
    br = v.shape[0]
    va = v[:, :, None]
    vb = v[:, None, :]
    ii3 = lax.broadcasted_iota(jnp.int32, (br, KP, KP), 1)
    kio3 = lax.broadcasted_iota(jnp.int32, (br, KP, KP), 2)
    cmp = (va > vb) | ((va == vb) & (ii3 < kio3))
    rank = jnp.sum(cmp.astype(jnp.int32), axis=1)
    oneh = rank[:, :, None] == kio3
    sv = jnp.sum(jnp.where(oneh, va, 0.0), axis=1)
    kio2 = lax.broadcasted_iota(jnp.int32, (br, KP), 1)
    sv = jnp.where(kio2 < K, sv, 0.0)
    rank_ref[...] = rank
    h = jnp.maximum(
        jnp.dot(sv, w_in_ref[...],
                preferred_element_type=jnp.float32) + b_in_ref[...], 0.0)
    e_ref[...] = jnp.dot(h, w_et_ref[...],
                         preferred_element_type=jnp.float32) + cvec[...]


def _sc_scatter_kernel(x_hbm, ci_hbm, rk_hbm, e_hbm, simu_hbm,
                       cibuf, rkbuf, ebuf, rows):
    wid = lax.axis_index("s") * 2 + lax.axis_index("c")
    base = wid * RPW
    pltpu.sync_copy(ci_hbm.at[pl.ds(base * KP, RPW * KP)], cibuf)
    pltpu.sync_copy(rk_hbm.at[pl.ds(base * KP, RPW * KP)], rkbuf)
    pltpu.sync_copy(e_hbm.at[pl.ds(base * KP, RPW * KP)], ebuf)
    ch = 32
    nch = RPW // ch

    def chunk_body(c, _):
        row0 = base + c * ch
        pltpu.sync_copy(x_hbm.at[pl.ds(row0 * D, ch * D)], rows)

        def row_body(r, _):
            gr = c * ch + r
            roff = r * D
            ebase = gr * KP
            for j in range(8):
                rv = rkbuf[pl.ds(ebase + j * 16, 16)]
                keep = rv < K
                ev = plsc.load_gather(ebuf, [rv + ebase])
                iv = cibuf[pl.ds(ebase + j * 16, 16)]
                iv = jnp.where(keep, iv + roff, 0)
                plsc.store_scatter(rows, [iv], ev, mask=keep)
            return 0

        lax.fori_loop(0, ch, row_body, 0, unroll=8)
        pltpu.sync_copy(rows, simu_hbm.at[pl.ds(row0 * D, ch * D)])
        return 0

    lax.fori_loop(0, nch, chunk_body, 0)


def _sample_kernel(x_ref, g_ref, wdec_ref, bdec_ref, out_ref, acc):
    i = pl.program_id(0)
    n = pl.num_programs(0)
    br = x_ref.shape[0]
    lane = lax.broadcasted_iota(jnp.int32, (br, D), 1)

    @pl.when(i == 0)
    def _():
        acc[...] = jnp.zeros_like(acc)

    y = x_ref[...] + g_ref[...]
    m2 = jnp.max(y, axis=1, keepdims=True)
    s = jnp.min(jnp.where(y == m2, lane, D), axis=1, keepdims=True)
    sf = s.astype(jnp.float32)
    acc[...] += jnp.sum(sf * wdec_ref[...], axis=0, keepdims=True)

    @pl.when(i == n - 1)
    def _():
        o = acc[...] + bdec_ref[...]
        nrm = jnp.sqrt(jnp.sum(o * o))
        out_ref[...] = o / jnp.maximum(nrm, 1e-12)


_SC_PARAMS = pltpu.CompilerParams(needs_layout_passes=False,
                                  use_tc_tiling_on_sc=False)
_SC_MESH = dict(core_axis_name="c", subcore_axis_name="s")


def kernel(input_data, code, W_in, b_in, W_code, b_code, W_enc, b_enc,
           W_dec, b_dec, d_constraint):
    del d_constraint
    f32 = jnp.float32
    i32 = jnp.int32

    w_in_pad = jnp.zeros((KP, H), f32).at[:K].set(W_in)
    w_et = jnp.zeros((H, KP), f32).at[:, :K].set(W_enc[:H])
    w_eb = jnp.zeros((H, KP), f32).at[:, :K].set(W_enc[H:])
    b_enc_pad = jnp.zeros((1, KP), f32).at[0, :K].set(b_enc)
    gumbel = jax.random.gumbel(jax.random.key(1234), (L, D), f32)
    x_flat = input_data.reshape(L * D)

    BRB = 256
    tkey, mm = pl.pallas_call(
        _bisect_kernel,
        grid=(L // BRB,),
        in_specs=[pl.BlockSpec((BRB, D), lambda i: (i, 0))],
        out_specs=[pl.BlockSpec((BRB, 1), lambda i: (i, 0)),
                   pl.BlockSpec((BRB, 1), lambda i: (i, 0))],
        out_shape=[jax.ShapeDtypeStruct((L, 1), i32),
                   jax.ShapeDtypeStruct((L, 1), i32)],
    )(input_data)

    cv_flat, ci_flat = pl.kernel(
        _sc_compact_kernel,
        out_type=[jax.ShapeDtypeStruct((L * KP,), f32),
                  jax.ShapeDtypeStruct((L * KP,), i32)],
        mesh=plsc.VectorSubcoreMesh(**_SC_MESH),
        scratch_types=[
            pltpu.VMEM((16 * D,), f32),
            pltpu.VMEM((RPW,), i32),
            pltpu.VMEM((RPW,), i32),
            pltpu.VMEM((RPW * KP,), f32),
            pltpu.VMEM((RPW * KP,), i32),
        ],
        compiler_params=_SC_PARAMS,
    )(x_flat, tkey.reshape(L), mm.reshape(L))
    cv = cv_flat.reshape(L, KP)

    BR2 = 64
    e, rank = pl.pallas_call(
        _rank_mlp_kernel,
        grid=(L // BR2,),
        in_specs=[
            pl.BlockSpec((BR2, KP), lambda i: (i, 0)),
            pl.BlockSpec((KP, H), lambda i: (0, 0)),
            pl.BlockSpec((1, H), lambda i: (0, 0)),
            pl.BlockSpec((1, CODE), lambda i: (0, 0)),
            pl.BlockSpec((CODE, H), lambda i: (0, 0)),
            pl.BlockSpec((1, H), lambda i: (0, 0)),
            pl.BlockSpec((H, KP), lambda i: (0, 0)),
            pl.BlockSpec((H, KP), lambda i: (0, 0)),
            pl.BlockSpec((1, KP), lambda i: (0, 0)),
        ],
        out_specs=[pl.BlockSpec((BR2, KP), lambda i: (i, 0)),
                   pl.BlockSpec((BR2, KP), lambda i: (i, 0))],
        out_shape=[jax.ShapeDtypeStruct((L, KP), f32),
                   jax.ShapeDtypeStruct((L, KP), i32)],
        scratch_shapes=[pltpu.VMEM((1, KP), f32)],
    )(cv, w_in_pad, b_in.reshape(1, H), code.reshape(1, CODE), W_code,
      b_code.reshape(1, H), w_et, w_eb, b_enc_pad)

    simu = pl.kernel(
        _sc_scatter_kernel,
        out_type=jax.ShapeDtypeStruct((L * D,), f32),
        mesh=plsc.VectorSubcoreMesh(**_SC_MESH),
        scratch_types=[
            pltpu.VMEM((RPW * KP,), i32),
            pltpu.VMEM((RPW * KP,), i32),
            pltpu.VMEM((RPW * KP,), f32),
            pltpu.VMEM((32 * D,), f32),
        ],
        compiler_params=_SC_PARAMS,
    )(x_flat, ci_flat, rank.reshape(L * KP), e.reshape(L * KP))
    simu = simu.reshape(L, D)

    BR3 = 256
    out = pl.pallas_call(
        _sample_kernel,
        grid=(L // BR3,),
        in_specs=[
            pl.BlockSpec((BR3, D), lambda i: (i, 0)),
            pl.BlockSpec((BR3, D), lambda i: (i, 0)),
            pl.BlockSpec((BR3, CODE), lambda i: (i, 0)),
            pl.BlockSpec((1, CODE), lambda i: (0, 0)),
        ],
        out_specs=pl.BlockSpec((1, CODE), lambda i: (0, 0)),
        out_shape=jax.ShapeDtypeStruct((1, CODE), f32),
        scratch_shapes=[pltpu.VMEM((1, CODE), f32)],
    )(simu, gumbel, W_dec, b_dec.reshape(1, CODE))

    return out.reshape(CODE)
